# Initial kernel scaffold; baseline (speedup 1.0000x reference)
import jax
import jax.numpy as jnp
from jax.experimental import pallas as pl


def kernel(x, embedding, pos_encoding):
    raise NotImplementedError("write your pallas kernel here")


# speedup vs baseline: 4.2152x; 4.2152x over previous
import functools

import jax
import jax.numpy as jnp
from jax import lax
from jax.experimental import pallas as pl
from jax.experimental.pallas import tpu as pltpu
from jax.experimental.pallas import tpu_sc as plsc

D = 32
T = 200
NC = 2
NS = 16
NW = NC * NS

ROWS_PER_CHUNK = 4
CHUNK = ROWS_PER_CHUNK * T
GATHER_SLICES = [(j * 128, 128) for j in range(6)] + [(768, 32)]


def _make_kernel(n_total):
  per_w = n_total // NW
  n_chunks = per_w // CHUNK
  mesh = plsc.VectorSubcoreMesh(
      core_axis_name="c", subcore_axis_name="s", num_cores=NC,
      num_subcores=NS)

  @functools.partial(
      pl.kernel,
      out_type=jax.ShapeDtypeStruct((n_total, D), jnp.float32),
      mesh=mesh,
      scratch_types=[
          pltpu.VMEM((T, D), jnp.float32),
          pltpu.VMEM((CHUNK,), jnp.int32),
          pltpu.VMEM((CHUNK, D), jnp.float32),
          pltpu.SemaphoreType.DMA,
      ],
      compiler_params=pltpu.CompilerParams(use_tc_tiling_on_sc=False),
  )
  def k(idx_hbm, emb_hbm, pos_hbm, out_hbm, pos_v, idx_v, rows_v, sem):
    wid = lax.axis_index("s") * NC + lax.axis_index("c")
    wbase = wid * per_w
    pltpu.sync_copy(pos_hbm, pos_v)

    def chunk_body(g, carry):
      base = wbase + g * CHUNK
      pltpu.sync_copy(idx_hbm.at[pl.ds(base, CHUNK)], idx_v)
      copies = [
          pltpu.async_copy(
              emb_hbm.at[idx_v.at[pl.ds(off, sz)]],
              rows_v.at[pl.ds(off, sz)], sem)
          for off, sz in GATHER_SLICES
      ]
      for c in copies:
        c.wait()

      def add_body(t, carry2):
        for d in range(D // 16):
          p = pos_v[t, pl.ds(d * 16, 16)]
          for r in range(ROWS_PER_CHUNK):
            row = r * T + t
            rows_v[row, pl.ds(d * 16, 16)] = (
                rows_v[row, pl.ds(d * 16, 16)] + p)
        return carry2

      lax.fori_loop(0, T, add_body, 0, unroll=2)
      pltpu.sync_copy(rows_v, out_hbm.at[pl.ds(base, CHUNK)])
      return carry

    lax.fori_loop(0, n_chunks, chunk_body, 0)

  return k


def kernel(x, embedding, pos_encoding):
  b, t = x.shape
  n_total = b * t
  out = _make_kernel(n_total)(x.reshape(n_total), embedding, pos_encoding)
  return out.reshape(b, t, D)


# speedup vs baseline: 4.9696x; 1.1789x over previous
import functools

import jax
import jax.numpy as jnp
from jax import lax
from jax.experimental import pallas as pl
from jax.experimental.pallas import tpu as pltpu
from jax.experimental.pallas import tpu_sc as plsc

D = 32
T = 200
NC = 2
NS = 16
NW = NC * NS

ROWS_PER_CHUNK = 4
CHUNK = ROWS_PER_CHUNK * T
NSLOT = 4
GATHER_SLICES = [(j * 128, 128) for j in range(6)] + [(768, 32)]


def _make_kernel(n_total):
  per_w = n_total // NW
  n_chunks = per_w // CHUNK
  n_groups = n_chunks // NSLOT
  mesh = plsc.VectorSubcoreMesh(
      core_axis_name="c", subcore_axis_name="s", num_cores=NC,
      num_subcores=NS)

  @functools.partial(
      pl.kernel,
      out_type=jax.ShapeDtypeStruct((n_total, D), jnp.float32),
      mesh=mesh,
      scratch_types=[
          pltpu.VMEM((T, D), jnp.float32),
          pltpu.VMEM((NSLOT, CHUNK), jnp.int32),
          pltpu.VMEM((NSLOT, CHUNK, D), jnp.float32),
          pltpu.SemaphoreType.DMA((NSLOT,)),
          pltpu.SemaphoreType.DMA((NSLOT,)),
      ],
      compiler_params=pltpu.CompilerParams(use_tc_tiling_on_sc=False),
  )
  def k(idx_hbm, emb_hbm, pos_hbm, out_hbm, pos_v, idx_v, rows_v, gsem,
        ssem):
    wid = lax.axis_index("s") * NC + lax.axis_index("c")
    wbase = wid * per_w
    pltpu.sync_copy(pos_hbm, pos_v)

    def load(g, s):
      base = wbase + g * CHUNK
      pltpu.sync_copy(idx_hbm.at[pl.ds(base, CHUNK)], idx_v.at[s])
      for off, sz in GATHER_SLICES:
        pltpu.async_copy(
            emb_hbm.at[idx_v.at[s, pl.ds(off, sz)]],
            rows_v.at[s, pl.ds(off, sz)], gsem.at[s])

    def drain_gathers(s):
      pltpu.make_async_copy(
          out_hbm.at[pl.ds(0, CHUNK)], rows_v.at[s], gsem.at[s]).wait()

    def drain_store(s):
      pltpu.make_async_copy(
          rows_v.at[s], out_hbm.at[pl.ds(0, CHUNK)], ssem.at[s]).wait()

    def finish(g, s):
      drain_gathers(s)

      def add_body(t, carry):
        for d in range(D // 16):
          p = pos_v[t, pl.ds(d * 16, 16)]
          for r in range(ROWS_PER_CHUNK):
            row = r * T + t
            rows_v[s, row, pl.ds(d * 16, 16)] = (
                rows_v[s, row, pl.ds(d * 16, 16)] + p)
        return carry

      lax.fori_loop(0, T, add_body, 0, unroll=2)
      base = wbase + g * CHUNK
      pltpu.async_copy(rows_v.at[s], out_hbm.at[pl.ds(base, CHUNK)],
                       ssem.at[s])

    load(0, 0)

    def group_body(p, carry):
      g0 = p * NSLOT
      for b in range(NSLOT):
        g = g0 + b
        nxt = g + 1
        s_nxt = (b + 1) % NSLOT
        if b == NSLOT - 1:
          @pl.when(p < n_groups - 1)
          def _():
            drain_store(s_nxt)
            load(nxt, s_nxt)
        else:
          @pl.when(p > 0)
          def _():
            drain_store(s_nxt)
          load(nxt, s_nxt)
        finish(g, b)
      return carry

    lax.fori_loop(0, n_groups, group_body, 0)
    for s in range(NSLOT):
      drain_store(s)

  return k


def kernel(x, embedding, pos_encoding):
  b, t = x.shape
  n_total = b * t
  out = _make_kernel(n_total)(x.reshape(n_total), embedding, pos_encoding)
  return out.reshape(b, t, D)
